# baseline (speedup 1.0000x reference)
import functools

import jax
import jax.numpy as jnp
from jax import lax
from jax.experimental import pallas as pl
from jax.experimental.pallas import tpu as pltpu
from jax.experimental.pallas import tpu_sc as plsc

_D = 128
_NC = 8
_NTRAIN = 100000
_K = 10

_RB = 6272
_GRID = 16
_NPAD = _RB * _GRID
_NVEC = _RB // 16


def _dense_kernel(te_ref, c_ref, cl_ref, w_ref, b_ref, y_ref, scal_ref):
    c = c_ref[...]
    gram = lax.dot_general(c, c, (((0,), (0,)), ((), ())),
                           preferred_element_type=jnp.float32)

    eye8 = jnp.eye(8, dtype=jnp.float32)
    aug = jnp.concatenate([gram, eye8], axis=1)
    rid = lax.broadcasted_iota(jnp.int32, (8, 16), 0)
    for j in range(8):
        pv = jnp.sum(aug[j:j + 1, j:j + 1])
        row = aug[j:j + 1, :] / pv
        fac = aug[:, j:j + 1]
        aug = jnp.where(rid == j, row, aug - fac * row)
    inv = aug[:, 8:]

    te = te_ref[...]
    a = jnp.dot(te, c, preferred_element_type=jnp.float32)
    m = lax.dot_general(c, w_ref[...], (((0,), (0,)), ((), ())),
                        preferred_element_type=jnp.float32)
    y = jnp.dot(jnp.dot(a, inv, preferred_element_type=jnp.float32), m,
                preferred_element_type=jnp.float32) + b_ref[...]
    y_ref[...] = y

    cm = cl_ref[:, 0, :]
    for s in range(1, 50):
        cm = cm + cl_ref[:, s, :]
    cm = cm * (1.0 / 50.0)

    cnorm = jnp.sqrt(jnp.sum(c * c, axis=0, keepdims=True))
    cn = c / jnp.maximum(cnorm, 1e-12)
    score = jnp.abs(jnp.dot(cm, cn, preferred_element_type=jnp.float32))
    sden = jnp.sqrt(jnp.sum(score * score, axis=0, keepdims=True))
    sn = score / jnp.maximum(sden, 1e-12)
    l1_old = jnp.sum(sn)
    g = lax.dot_general(sn, sn, (((0,), (0,)), ((), ())),
                        preferred_element_type=jnp.float32)
    r8 = lax.broadcasted_iota(jnp.int32, (8, 8), 0)
    c8 = lax.broadcasted_iota(jnp.int32, (8, 8), 1)
    l2_old = jnp.sum(jnp.where(r8 == c8, 0.0, g))
    l2_new = jnp.sum(jnp.where(r8 == c8, 0.0, gram)) * (1.0 / 64.0)

    scal_ref[...] = jnp.concatenate(
        [jnp.full((1, 1), l1_old, jnp.float32),
         jnp.full((1, 1), l2_old, jnp.float32),
         jnp.full((1, 1), l2_new, jnp.float32),
         jnp.zeros((1, 5), jnp.float32)], axis=1)


def _dist_kernel(te_ref, c_ref, dist_ref, dots_ref):
    i = pl.program_id(0)
    blk = te_ref[...]
    c = c_ref[...]
    dots = lax.dot_general(c, blk, (((0,), (1,)), ((), ())),
                           preferred_element_type=jnp.float32)
    sq = blk * blk
    ones = jnp.ones((1, _D), jnp.float32)
    norms = lax.dot_general(ones, sq, (((1,), (1,)), ((), ())),
                            preferred_element_type=jnp.float32)
    cn2 = jnp.sum(c * c, axis=0, keepdims=True)
    dist = norms - 2.0 * dots + cn2.reshape(_NC, 1)
    colid = i * _RB + lax.broadcasted_iota(jnp.int32, (_NC, _RB), 1)
    valid = colid < _NTRAIN
    dist_ref[...] = jnp.where(valid, dist, jnp.inf)
    dots_ref[...] = jnp.where(valid, dots, 0.0)


def _merge_sorted16(bk, bv, nk, nv):
    nk = lax.rev(nk, (0,))
    nv = lax.rev(nv, (0,))
    m = bk <= nk
    lk = jnp.where(m, bk, nk)
    lv = jnp.where(m, bv, nv)
    sk, sv = plsc.sort_key_val(lk, lv)
    return sk, sv


def _sc_phase1_body(dist_hbm, dots_hbm, ck_hbm, cv_hbm, kbuf, vbuf, pbk, pbv):
    core = lax.axis_index("c")
    sub = lax.axis_index("s")
    base = sub * _RB

    inf16 = jnp.full((16,), jnp.inf, jnp.float32)
    zero16 = jnp.zeros((16,), jnp.float32)

    for j in range(4):
        cglob = core * 4 + j
        pltpu.sync_copy(dist_hbm.at[cglob, pl.ds(base, _RB)], kbuf.at[j])
        pltpu.sync_copy(dots_hbm.at[cglob, pl.ds(base, _RB)], vbuf.at[j])

    for j in range(4):
        def body(it, carry, _j=j):
            bk, bv = carry
            nk = kbuf[_j, pl.ds(it * 16, 16)]
            thr = jnp.max(bk)

            def do_merge(args):
                bk, bv = args
                nv = vbuf[_j, pl.ds(it * 16, 16)]
                sk, sv = plsc.sort_key_val(nk, nv)
                return _merge_sorted16(bk, bv, sk, sv)

            return lax.cond(jnp.any(nk < thr), do_merge, lambda a: a,
                            (bk, bv))

        bk, bv = lax.fori_loop(0, _NVEC, body, (inf16, zero16))
        pbk[j, :] = bk
        pbv[j, :] = bv

    for j in range(4):
        pltpu.sync_copy(pbk.at[j], ck_hbm.at[core, j, sub])
        pltpu.sync_copy(pbv.at[j], cv_hbm.at[core, j, sub])


def _sc_phase2_body(ck_hbm, cv_hbm, out_hbm, mkbuf, mvbuf, obuf):
    core = lax.axis_index("c")
    sub = lax.axis_index("s")
    lanes = lax.iota(jnp.int32, 16)
    inf16 = jnp.full((16,), jnp.inf, jnp.float32)
    zero16 = jnp.zeros((16,), jnp.float32)

    @pl.when(sub < 4)
    def _():
        pltpu.sync_copy(ck_hbm.at[core, sub], mkbuf)
        pltpu.sync_copy(cv_hbm.at[core, sub], mvbuf)
        bk, bv = inf16, zero16
        for t in range(16):
            bk, bv = _merge_sorted16(bk, bv, mkbuf[t, :], mvbuf[t, :])
        top10 = jnp.sum(jnp.where(lanes < _K, bv, 0.0))
        obuf[...] = jnp.full((16,), top10, jnp.float32)
        pltpu.sync_copy(obuf, out_hbm.at[core * 4 + sub])


@functools.lru_cache(maxsize=1)
def _sc_topk():
    mesh = plsc.VectorSubcoreMesh(core_axis_name="c", subcore_axis_name="s")
    cand_t = jax.ShapeDtypeStruct((2, 4, 16, 16), jnp.float32)
    phase1 = pl.kernel(
        _sc_phase1_body,
        mesh=mesh,
        out_type=[cand_t, cand_t],
        compiler_params=pltpu.CompilerParams(needs_layout_passes=False),
        scratch_types=[
            pltpu.VMEM((4, _RB), jnp.float32),
            pltpu.VMEM((4, _RB), jnp.float32),
            pltpu.VMEM((4, 16), jnp.float32),
            pltpu.VMEM((4, 16), jnp.float32),
        ],
    )
    phase2 = pl.kernel(
        _sc_phase2_body,
        mesh=mesh,
        out_type=jax.ShapeDtypeStruct((_NC, 16), jnp.float32),
        compiler_params=pltpu.CompilerParams(needs_layout_passes=False),
        scratch_types=[
            pltpu.VMEM((16, 16), jnp.float32),
            pltpu.VMEM((16, 16), jnp.float32),
            pltpu.VMEM((16,), jnp.float32),
        ],
    )

    def run(dist, dots):
        ck, cv = phase1(dist, dots)
        return phase2(ck, cv)

    return run


def kernel(train_embedding, concept, clusters, train_embeddings, W_hx, b_hx):
    y_pred, scal = pl.pallas_call(
        _dense_kernel,
        out_shape=[
            jax.ShapeDtypeStruct((1024, 10), jnp.float32),
            jax.ShapeDtypeStruct((1, 8), jnp.float32),
        ],
    )(train_embedding, concept, clusters, W_hx, b_hx.reshape(1, 10))

    dist, dots = pl.pallas_call(
        _dist_kernel,
        grid=(_GRID,),
        in_specs=[
            pl.BlockSpec((_RB, _D), lambda i: (i, 0)),
            pl.BlockSpec((_D, _NC), lambda i: (0, 0)),
        ],
        out_specs=[
            pl.BlockSpec((_NC, _RB), lambda i: (0, i)),
            pl.BlockSpec((_NC, _RB), lambda i: (0, i)),
        ],
        out_shape=[
            jax.ShapeDtypeStruct((_NC, _NPAD), jnp.float32),
            jax.ShapeDtypeStruct((_NC, _NPAD), jnp.float32),
        ],
    )(train_embeddings, concept)

    knn = _sc_topk()(dist, dots)

    l1_new = jnp.sum(knn[:, 0]) * (1.0 / (_K * _NC))
    return (y_pred, scal[0, 0], scal[0, 1], l1_new, scal[0, 2])


# speedup vs baseline: 1.8883x; 1.8883x over previous
import functools

import jax
import jax.numpy as jnp
from jax import lax
from jax.experimental import pallas as pl
from jax.experimental.pallas import tpu as pltpu
from jax.experimental.pallas import tpu_sc as plsc

_D = 128
_NC = 8
_NTRAIN = 100000
_K = 10

_RB = 6272
_GRID = 16
_NPAD = _RB * _GRID
_NVEC = _RB // 16


def _dense_kernel(te_ref, c_ref, cl_ref, w_ref, b_ref, y_ref, scal_ref):
    c = c_ref[...]
    gram = lax.dot_general(c, c, (((0,), (0,)), ((), ())),
                           preferred_element_type=jnp.float32)

    eye8 = jnp.eye(8, dtype=jnp.float32)
    aug = jnp.concatenate([gram, eye8], axis=1)
    rid = lax.broadcasted_iota(jnp.int32, (8, 16), 0)
    for j in range(8):
        pv = jnp.sum(aug[j:j + 1, j:j + 1])
        row = aug[j:j + 1, :] / pv
        fac = aug[:, j:j + 1]
        aug = jnp.where(rid == j, row, aug - fac * row)
    inv = aug[:, 8:]

    te = te_ref[...]
    a = jnp.dot(te, c, preferred_element_type=jnp.float32)
    m = lax.dot_general(c, w_ref[...], (((0,), (0,)), ((), ())),
                        preferred_element_type=jnp.float32)
    y = jnp.dot(jnp.dot(a, inv, preferred_element_type=jnp.float32), m,
                preferred_element_type=jnp.float32) + b_ref[...]
    y_ref[...] = y

    cm = cl_ref[:, 0, :]
    for s in range(1, 50):
        cm = cm + cl_ref[:, s, :]
    cm = cm * (1.0 / 50.0)

    cnorm = jnp.sqrt(jnp.sum(c * c, axis=0, keepdims=True))
    cn = c / jnp.maximum(cnorm, 1e-12)
    score = jnp.abs(jnp.dot(cm, cn, preferred_element_type=jnp.float32))
    sden = jnp.sqrt(jnp.sum(score * score, axis=0, keepdims=True))
    sn = score / jnp.maximum(sden, 1e-12)
    l1_old = jnp.sum(sn)
    g = lax.dot_general(sn, sn, (((0,), (0,)), ((), ())),
                        preferred_element_type=jnp.float32)
    r8 = lax.broadcasted_iota(jnp.int32, (8, 8), 0)
    c8 = lax.broadcasted_iota(jnp.int32, (8, 8), 1)
    l2_old = jnp.sum(jnp.where(r8 == c8, 0.0, g))
    l2_new = jnp.sum(jnp.where(r8 == c8, 0.0, gram)) * (1.0 / 64.0)

    scal_ref[...] = jnp.concatenate(
        [jnp.full((1, 1), l1_old, jnp.float32),
         jnp.full((1, 1), l2_old, jnp.float32),
         jnp.full((1, 1), l2_new, jnp.float32),
         jnp.zeros((1, 5), jnp.float32)], axis=1)


def _dist_kernel(te_ref, c_ref, dist_ref, dots_ref):
    i = pl.program_id(0)
    blk = te_ref[...]
    c = c_ref[...]
    dots = lax.dot_general(c, blk, (((0,), (1,)), ((), ())),
                           preferred_element_type=jnp.float32)
    sq = blk * blk
    ones = jnp.ones((1, _D), jnp.float32)
    norms = lax.dot_general(ones, sq, (((1,), (1,)), ((), ())),
                            preferred_element_type=jnp.float32)
    cn2 = jnp.sum(c * c, axis=0, keepdims=True)
    dist = norms - 2.0 * dots + cn2.reshape(_NC, 1)
    colid = i * _RB + lax.broadcasted_iota(jnp.int32, (_NC, _RB), 1)
    valid = colid < _NTRAIN
    dist_ref[...] = jnp.where(valid, dist, jnp.inf)
    dots_ref[...] = jnp.where(valid, dots, 0.0)


def _merge_sorted16(bk, bv, nk, nv):
    nk = lax.rev(nk, (0,))
    nv = lax.rev(nv, (0,))
    m = bk <= nk
    lk = jnp.where(m, bk, nk)
    lv = jnp.where(m, bv, nv)
    sk, sv = plsc.sort_key_val(lk, lv)
    return sk, sv


def _sc_phase1_body(dist_hbm, dots_hbm, ck_hbm, cv_hbm, kbuf, vbuf, pbk, pbv):
    core = lax.axis_index("c")
    sub = lax.axis_index("s")
    base = sub * _RB

    inf16 = jnp.full((16,), jnp.inf, jnp.float32)
    zero16 = jnp.zeros((16,), jnp.float32)

    for j in range(4):
        cglob = core * 4 + j
        pltpu.sync_copy(dist_hbm.at[cglob, pl.ds(base, _RB)], kbuf.at[j])
        pltpu.sync_copy(dots_hbm.at[cglob, pl.ds(base, _RB)], vbuf.at[j])

    for j in range(4):
        def body(it, carry, _j=j):
            new = []
            for s in range(4):
                bk, bv = carry[2 * s], carry[2 * s + 1]
                off = (it * 4 + s) * 16
                nk = kbuf[_j, pl.ds(off, 16)]
                nv = vbuf[_j, pl.ds(off, 16)]
                sk, sv = plsc.sort_key_val(nk, nv)
                bk, bv = _merge_sorted16(bk, bv, sk, sv)
                new += [bk, bv]
            return tuple(new)

        st = lax.fori_loop(0, _NVEC // 4, body, (inf16, zero16) * 4)
        bk0, bv0 = _merge_sorted16(st[0], st[1], st[2], st[3])
        bk1, bv1 = _merge_sorted16(st[4], st[5], st[6], st[7])
        bk, bv = _merge_sorted16(bk0, bv0, bk1, bv1)
        pbk[j, :] = bk
        pbv[j, :] = bv

    for j in range(4):
        pltpu.sync_copy(pbk.at[j], ck_hbm.at[core, j, sub])
        pltpu.sync_copy(pbv.at[j], cv_hbm.at[core, j, sub])


def _sc_phase2_body(ck_hbm, cv_hbm, out_hbm, mkbuf, mvbuf, obuf):
    core = lax.axis_index("c")
    sub = lax.axis_index("s")
    lanes = lax.iota(jnp.int32, 16)
    inf16 = jnp.full((16,), jnp.inf, jnp.float32)
    zero16 = jnp.zeros((16,), jnp.float32)

    @pl.when(sub < 4)
    def _():
        pltpu.sync_copy(ck_hbm.at[core, sub], mkbuf)
        pltpu.sync_copy(cv_hbm.at[core, sub], mvbuf)
        bk, bv = inf16, zero16
        for t in range(16):
            bk, bv = _merge_sorted16(bk, bv, mkbuf[t, :], mvbuf[t, :])
        top10 = jnp.sum(jnp.where(lanes < _K, bv, 0.0))
        obuf[...] = jnp.full((16,), top10, jnp.float32)
        pltpu.sync_copy(obuf, out_hbm.at[core * 4 + sub])


@functools.lru_cache(maxsize=1)
def _sc_topk():
    mesh = plsc.VectorSubcoreMesh(core_axis_name="c", subcore_axis_name="s")
    cand_t = jax.ShapeDtypeStruct((2, 4, 16, 16), jnp.float32)
    phase1 = pl.kernel(
        _sc_phase1_body,
        mesh=mesh,
        out_type=[cand_t, cand_t],
        compiler_params=pltpu.CompilerParams(needs_layout_passes=False),
        scratch_types=[
            pltpu.VMEM((4, _RB), jnp.float32),
            pltpu.VMEM((4, _RB), jnp.float32),
            pltpu.VMEM((4, 16), jnp.float32),
            pltpu.VMEM((4, 16), jnp.float32),
        ],
    )
    phase2 = pl.kernel(
        _sc_phase2_body,
        mesh=mesh,
        out_type=jax.ShapeDtypeStruct((_NC, 16), jnp.float32),
        compiler_params=pltpu.CompilerParams(needs_layout_passes=False),
        scratch_types=[
            pltpu.VMEM((16, 16), jnp.float32),
            pltpu.VMEM((16, 16), jnp.float32),
            pltpu.VMEM((16,), jnp.float32),
        ],
    )

    def run(dist, dots):
        ck, cv = phase1(dist, dots)
        return phase2(ck, cv)

    return run


def kernel(train_embedding, concept, clusters, train_embeddings, W_hx, b_hx):
    y_pred, scal = pl.pallas_call(
        _dense_kernel,
        out_shape=[
            jax.ShapeDtypeStruct((1024, 10), jnp.float32),
            jax.ShapeDtypeStruct((1, 8), jnp.float32),
        ],
    )(train_embedding, concept, clusters, W_hx, b_hx.reshape(1, 10))

    dist, dots = pl.pallas_call(
        _dist_kernel,
        grid=(_GRID,),
        in_specs=[
            pl.BlockSpec((_RB, _D), lambda i: (i, 0)),
            pl.BlockSpec((_D, _NC), lambda i: (0, 0)),
        ],
        out_specs=[
            pl.BlockSpec((_NC, _RB), lambda i: (0, i)),
            pl.BlockSpec((_NC, _RB), lambda i: (0, i)),
        ],
        out_shape=[
            jax.ShapeDtypeStruct((_NC, _NPAD), jnp.float32),
            jax.ShapeDtypeStruct((_NC, _NPAD), jnp.float32),
        ],
    )(train_embeddings, concept)

    knn = _sc_topk()(dist, dots)

    l1_new = jnp.sum(knn[:, 0]) * (1.0 / (_K * _NC))
    return (y_pred, scal[0, 0], scal[0, 1], l1_new, scal[0, 2])


# speedup vs baseline: 2.0074x; 1.0631x over previous
import functools

import jax
import jax.numpy as jnp
from jax import lax
from jax.experimental import pallas as pl
from jax.experimental.pallas import tpu as pltpu
from jax.experimental.pallas import tpu_sc as plsc

_D = 128
_NC = 8
_NTRAIN = 100000
_K = 10

_RB = 6272
_GRID = 16
_NPAD = _RB * _GRID
_NVEC = _RB // 16


def _dense_kernel(te_ref, c_ref, cl_ref, w_ref, b_ref, y_ref, scal_ref):
    c = c_ref[...]
    gram = lax.dot_general(c, c, (((0,), (0,)), ((), ())),
                           preferred_element_type=jnp.float32)

    eye8 = jnp.eye(8, dtype=jnp.float32)
    aug = jnp.concatenate([gram, eye8], axis=1)
    rid = lax.broadcasted_iota(jnp.int32, (8, 16), 0)
    for j in range(8):
        pv = jnp.sum(aug[j:j + 1, j:j + 1])
        row = aug[j:j + 1, :] / pv
        fac = aug[:, j:j + 1]
        aug = jnp.where(rid == j, row, aug - fac * row)
    inv = aug[:, 8:]

    te = te_ref[...]
    a = jnp.dot(te, c, preferred_element_type=jnp.float32)
    m = lax.dot_general(c, w_ref[...], (((0,), (0,)), ((), ())),
                        preferred_element_type=jnp.float32)
    y = jnp.dot(jnp.dot(a, inv, preferred_element_type=jnp.float32), m,
                preferred_element_type=jnp.float32) + b_ref[...]
    y_ref[...] = y

    cm = cl_ref[:, 0, :]
    for s in range(1, 50):
        cm = cm + cl_ref[:, s, :]
    cm = cm * (1.0 / 50.0)

    cnorm = jnp.sqrt(jnp.sum(c * c, axis=0, keepdims=True))
    cn = c / jnp.maximum(cnorm, 1e-12)
    score = jnp.abs(jnp.dot(cm, cn, preferred_element_type=jnp.float32))
    sden = jnp.sqrt(jnp.sum(score * score, axis=0, keepdims=True))
    sn = score / jnp.maximum(sden, 1e-12)
    l1_old = jnp.sum(sn)
    g = lax.dot_general(sn, sn, (((0,), (0,)), ((), ())),
                        preferred_element_type=jnp.float32)
    r8 = lax.broadcasted_iota(jnp.int32, (8, 8), 0)
    c8 = lax.broadcasted_iota(jnp.int32, (8, 8), 1)
    l2_old = jnp.sum(jnp.where(r8 == c8, 0.0, g))
    l2_new = jnp.sum(jnp.where(r8 == c8, 0.0, gram)) * (1.0 / 64.0)

    scal_ref[...] = jnp.concatenate(
        [jnp.full((1, 1), l1_old, jnp.float32),
         jnp.full((1, 1), l2_old, jnp.float32),
         jnp.full((1, 1), l2_new, jnp.float32),
         jnp.zeros((1, 5), jnp.float32)], axis=1)


def _dist_kernel(te_ref, c_ref, norm_ref, dots_ref):
    i = pl.program_id(0)
    blk = te_ref[...]
    c = c_ref[...]
    dots = lax.dot_general(c, blk, (((0,), (1,)), ((), ())),
                           preferred_element_type=jnp.float32)
    sq = blk * blk
    ones = jnp.ones((1, _D), jnp.float32)
    norms = lax.dot_general(ones, sq, (((1,), (1,)), ((), ())),
                            preferred_element_type=jnp.float32)
    colid = i * _RB + lax.broadcasted_iota(jnp.int32, (_NC, _RB), 1)
    valid = colid < _NTRAIN
    norm_ref[...] = jnp.where(valid[:1], norms, jnp.inf)
    dots_ref[...] = jnp.where(valid, dots, 0.0)


def _merge_sorted16(bk, bv, nk, nv):
    nk = lax.rev(nk, (0,))
    nv = lax.rev(nv, (0,))
    m = bk <= nk
    lk = jnp.where(m, bk, nk)
    lv = jnp.where(m, bv, nv)
    sk, sv = plsc.sort_key_val(lk, lv)
    return sk, sv


def _sc_phase1_body(norm_hbm, dots_hbm, ck_hbm, cv_hbm, nbuf, vbuf, pbk, pbv):
    core = lax.axis_index("c")
    sub = lax.axis_index("s")
    base = sub * _RB

    inf16 = jnp.full((16,), jnp.inf, jnp.float32)
    zero16 = jnp.zeros((16,), jnp.float32)

    pltpu.sync_copy(norm_hbm.at[0, pl.ds(base, _RB)], nbuf)
    for j in range(4):
        cglob = core * 4 + j
        pltpu.sync_copy(dots_hbm.at[cglob, pl.ds(base, _RB)], vbuf.at[j])

    for j in range(4):
        def body(it, carry, _j=j):
            new = []
            for s in range(8):
                bk, bv = carry[2 * s], carry[2 * s + 1]
                off = (it * 8 + s) * 16
                nv = vbuf[_j, pl.ds(off, 16)]
                nk = nbuf[pl.ds(off, 16)] - 2.0 * nv
                sk, sv = plsc.sort_key_val(nk, nv)
                bk, bv = _merge_sorted16(bk, bv, sk, sv)
                new += [bk, bv]
            return tuple(new)

        st = lax.fori_loop(0, _NVEC // 8, body, (inf16, zero16) * 8)
        m = []
        for s in range(4):
            m += list(_merge_sorted16(st[4 * s], st[4 * s + 1],
                                      st[4 * s + 2], st[4 * s + 3]))
        bk0, bv0 = _merge_sorted16(m[0], m[1], m[2], m[3])
        bk1, bv1 = _merge_sorted16(m[4], m[5], m[6], m[7])
        bk, bv = _merge_sorted16(bk0, bv0, bk1, bv1)
        pbk[j, :] = bk
        pbv[j, :] = bv

    for j in range(4):
        pltpu.sync_copy(pbk.at[j], ck_hbm.at[core, j, sub])
        pltpu.sync_copy(pbv.at[j], cv_hbm.at[core, j, sub])


def _sc_phase2_body(ck_hbm, cv_hbm, out_hbm, mkbuf, mvbuf, obuf):
    core = lax.axis_index("c")
    sub = lax.axis_index("s")
    lanes = lax.iota(jnp.int32, 16)
    inf16 = jnp.full((16,), jnp.inf, jnp.float32)
    zero16 = jnp.zeros((16,), jnp.float32)

    @pl.when(sub < 4)
    def _():
        pltpu.sync_copy(ck_hbm.at[core, sub], mkbuf)
        pltpu.sync_copy(cv_hbm.at[core, sub], mvbuf)
        bk, bv = inf16, zero16
        for t in range(16):
            bk, bv = _merge_sorted16(bk, bv, mkbuf[t, :], mvbuf[t, :])
        top10 = jnp.sum(jnp.where(lanes < _K, bv, 0.0))
        obuf[...] = jnp.full((16,), top10, jnp.float32)
        pltpu.sync_copy(obuf, out_hbm.at[core * 4 + sub])


@functools.lru_cache(maxsize=1)
def _sc_topk():
    mesh = plsc.VectorSubcoreMesh(core_axis_name="c", subcore_axis_name="s")
    cand_t = jax.ShapeDtypeStruct((2, 4, 16, 16), jnp.float32)
    phase1 = pl.kernel(
        _sc_phase1_body,
        mesh=mesh,
        out_type=[cand_t, cand_t],
        compiler_params=pltpu.CompilerParams(needs_layout_passes=False),
        scratch_types=[
            pltpu.VMEM((_RB,), jnp.float32),
            pltpu.VMEM((4, _RB), jnp.float32),
            pltpu.VMEM((4, 16), jnp.float32),
            pltpu.VMEM((4, 16), jnp.float32),
        ],
    )
    phase2 = pl.kernel(
        _sc_phase2_body,
        mesh=mesh,
        out_type=jax.ShapeDtypeStruct((_NC, 16), jnp.float32),
        compiler_params=pltpu.CompilerParams(needs_layout_passes=False),
        scratch_types=[
            pltpu.VMEM((16, 16), jnp.float32),
            pltpu.VMEM((16, 16), jnp.float32),
            pltpu.VMEM((16,), jnp.float32),
        ],
    )

    def run(dist, dots):
        ck, cv = phase1(dist, dots)
        return phase2(ck, cv)

    return run


def kernel(train_embedding, concept, clusters, train_embeddings, W_hx, b_hx):
    y_pred, scal = pl.pallas_call(
        _dense_kernel,
        out_shape=[
            jax.ShapeDtypeStruct((1024, 10), jnp.float32),
            jax.ShapeDtypeStruct((1, 8), jnp.float32),
        ],
    )(train_embedding, concept, clusters, W_hx, b_hx.reshape(1, 10))

    dist, dots = pl.pallas_call(
        _dist_kernel,
        grid=(_GRID,),
        in_specs=[
            pl.BlockSpec((_RB, _D), lambda i: (i, 0)),
            pl.BlockSpec((_D, _NC), lambda i: (0, 0)),
        ],
        out_specs=[
            pl.BlockSpec((1, _RB), lambda i: (0, i)),
            pl.BlockSpec((_NC, _RB), lambda i: (0, i)),
        ],
        out_shape=[
            jax.ShapeDtypeStruct((1, _NPAD), jnp.float32),
            jax.ShapeDtypeStruct((_NC, _NPAD), jnp.float32),
        ],
    )(train_embeddings, concept)

    knn = _sc_topk()(dist, dots)

    l1_new = jnp.sum(knn[:, 0]) * (1.0 / (_K * _NC))
    return (y_pred, scal[0, 0], scal[0, 1], l1_new, scal[0, 2])


# speedup vs baseline: 2.0121x; 1.0023x over previous
import functools

import jax
import jax.numpy as jnp
from jax import lax
from jax.experimental import pallas as pl
from jax.experimental.pallas import tpu as pltpu
from jax.experimental.pallas import tpu_sc as plsc

_D = 128
_NC = 8
_NTRAIN = 100000
_K = 10

_RB = 6272
_GRID = 16
_NPAD = _RB * _GRID
_NVEC = _RB // 16


def _tc_kernel(te_s_ref, c_ref, te_ref, cl_ref, w_ref, b_ref,
               norm_ref, dots_ref, y_ref, scal_ref):
    i = pl.program_id(0)
    blk = te_s_ref[...]
    cc = c_ref[...]
    dots = lax.dot_general(cc, blk, (((0,), (1,)), ((), ())),
                           preferred_element_type=jnp.float32)
    sq = blk * blk
    ones = jnp.ones((1, _D), jnp.float32)
    norms = lax.dot_general(ones, sq, (((1,), (1,)), ((), ())),
                            preferred_element_type=jnp.float32)
    colid = i * _RB + lax.broadcasted_iota(jnp.int32, (_NC, _RB), 1)
    valid = colid < _NTRAIN
    norm_ref[...] = jnp.where(valid[:1], norms, jnp.inf)
    dots_ref[...] = jnp.where(valid, dots, 0.0)

    @pl.when(i == 0)
    def _():
        _dense_stage(te_ref, c_ref, cl_ref, w_ref, b_ref, y_ref, scal_ref)


def _dense_stage(te_ref, c_ref, cl_ref, w_ref, b_ref, y_ref, scal_ref):
    c = c_ref[...]
    gram = lax.dot_general(c, c, (((0,), (0,)), ((), ())),
                           preferred_element_type=jnp.float32)

    eye8 = jnp.eye(8, dtype=jnp.float32)
    aug = jnp.concatenate([gram, eye8], axis=1)
    rid = lax.broadcasted_iota(jnp.int32, (8, 16), 0)
    for j in range(8):
        pv = jnp.sum(aug[j:j + 1, j:j + 1])
        row = aug[j:j + 1, :] / pv
        fac = aug[:, j:j + 1]
        aug = jnp.where(rid == j, row, aug - fac * row)
    inv = aug[:, 8:]

    te = te_ref[...]
    a = jnp.dot(te, c, preferred_element_type=jnp.float32)
    m = lax.dot_general(c, w_ref[...], (((0,), (0,)), ((), ())),
                        preferred_element_type=jnp.float32)
    y = jnp.dot(jnp.dot(a, inv, preferred_element_type=jnp.float32), m,
                preferred_element_type=jnp.float32) + b_ref[...]
    y_ref[...] = y

    cm = cl_ref[:, 0, :]
    for s in range(1, 50):
        cm = cm + cl_ref[:, s, :]
    cm = cm * (1.0 / 50.0)

    cnorm = jnp.sqrt(jnp.sum(c * c, axis=0, keepdims=True))
    cn = c / jnp.maximum(cnorm, 1e-12)
    score = jnp.abs(jnp.dot(cm, cn, preferred_element_type=jnp.float32))
    sden = jnp.sqrt(jnp.sum(score * score, axis=0, keepdims=True))
    sn = score / jnp.maximum(sden, 1e-12)
    l1_old = jnp.sum(sn)
    g = lax.dot_general(sn, sn, (((0,), (0,)), ((), ())),
                        preferred_element_type=jnp.float32)
    r8 = lax.broadcasted_iota(jnp.int32, (8, 8), 0)
    c8 = lax.broadcasted_iota(jnp.int32, (8, 8), 1)
    l2_old = jnp.sum(jnp.where(r8 == c8, 0.0, g))
    l2_new = jnp.sum(jnp.where(r8 == c8, 0.0, gram)) * (1.0 / 64.0)

    scal_ref[...] = jnp.concatenate(
        [jnp.full((1, 1), l1_old, jnp.float32),
         jnp.full((1, 1), l2_old, jnp.float32),
         jnp.full((1, 1), l2_new, jnp.float32),
         jnp.zeros((1, 5), jnp.float32)], axis=1)


def _merge_sorted16(bk, bv, nk, nv):
    nk = lax.rev(nk, (0,))
    nv = lax.rev(nv, (0,))
    m = bk <= nk
    lk = jnp.where(m, bk, nk)
    lv = jnp.where(m, bv, nv)
    sk, sv = plsc.sort_key_val(lk, lv)
    return sk, sv


def _sc_topk_body(norm_hbm, dots_hbm, out_hbm, ck_hbm, cv_hbm,
                  nbuf, vbuf, pbk, pbv, mkbuf, mvbuf, obuf):
    core = lax.axis_index("c")
    sub = lax.axis_index("s")
    base = sub * _RB

    lanes = lax.iota(jnp.int32, 16)
    inf16 = jnp.full((16,), jnp.inf, jnp.float32)
    zero16 = jnp.zeros((16,), jnp.float32)

    pltpu.sync_copy(norm_hbm.at[0, pl.ds(base, _RB)], nbuf)
    for j in range(4):
        cglob = core * 4 + j
        pltpu.sync_copy(dots_hbm.at[cglob, pl.ds(base, _RB)], vbuf.at[j])

    for j in range(4):
        def body(it, carry, _j=j):
            new = []
            for s in range(8):
                bk, bv = carry[2 * s], carry[2 * s + 1]
                off = (it * 8 + s) * 16
                nv = vbuf[_j, pl.ds(off, 16)]
                nk = nbuf[pl.ds(off, 16)] - 2.0 * nv
                sk, sv = plsc.sort_key_val(nk, nv)
                bk, bv = _merge_sorted16(bk, bv, sk, sv)
                new += [bk, bv]
            return tuple(new)

        st = lax.fori_loop(0, _NVEC // 8, body, (inf16, zero16) * 8)
        m = []
        for s in range(4):
            m += list(_merge_sorted16(st[4 * s], st[4 * s + 1],
                                      st[4 * s + 2], st[4 * s + 3]))
        bk0, bv0 = _merge_sorted16(m[0], m[1], m[2], m[3])
        bk1, bv1 = _merge_sorted16(m[4], m[5], m[6], m[7])
        bk, bv = _merge_sorted16(bk0, bv0, bk1, bv1)
        pbk[j, :] = bk
        pbv[j, :] = bv

    for j in range(4):
        pltpu.sync_copy(pbk.at[j], ck_hbm.at[core, j, sub])
        pltpu.sync_copy(pbv.at[j], cv_hbm.at[core, j, sub])

    plsc.subcore_barrier()

    @pl.when(sub < 4)
    def _():
        pltpu.sync_copy(ck_hbm.at[core, sub], mkbuf)
        pltpu.sync_copy(cv_hbm.at[core, sub], mvbuf)
        bk, bv = inf16, zero16
        for t in range(16):
            bk, bv = _merge_sorted16(bk, bv, mkbuf[t, :], mvbuf[t, :])
        top10 = jnp.sum(jnp.where(lanes < _K, bv, 0.0))
        obuf[...] = jnp.full((16,), top10, jnp.float32)
        pltpu.sync_copy(obuf, out_hbm.at[core * 4 + sub])


@functools.lru_cache(maxsize=1)
def _sc_topk():
    cand_t = jax.ShapeDtypeStruct((2, 4, 16, 16), jnp.float32)
    return pl.kernel(
        _sc_topk_body,
        mesh=plsc.VectorSubcoreMesh(core_axis_name="c", subcore_axis_name="s"),
        out_type=[jax.ShapeDtypeStruct((_NC, 16), jnp.float32),
                  cand_t, cand_t],
        compiler_params=pltpu.CompilerParams(needs_layout_passes=False),
        scratch_types=[
            pltpu.VMEM((_RB,), jnp.float32),
            pltpu.VMEM((4, _RB), jnp.float32),
            pltpu.VMEM((4, 16), jnp.float32),
            pltpu.VMEM((4, 16), jnp.float32),
            pltpu.VMEM((16, 16), jnp.float32),
            pltpu.VMEM((16, 16), jnp.float32),
            pltpu.VMEM((16,), jnp.float32),
        ],
    )


def kernel(train_embedding, concept, clusters, train_embeddings, W_hx, b_hx):
    norms, dots, y_pred, scal = pl.pallas_call(
        _tc_kernel,
        grid=(_GRID,),
        in_specs=[
            pl.BlockSpec((_RB, _D), lambda i: (i, 0)),
            pl.BlockSpec((_D, _NC), lambda i: (0, 0)),
            pl.BlockSpec((1024, _D), lambda i: (0, 0)),
            pl.BlockSpec((100, 50, _D), lambda i: (0, 0, 0)),
            pl.BlockSpec((_D, 10), lambda i: (0, 0)),
            pl.BlockSpec((1, 10), lambda i: (0, 0)),
        ],
        out_specs=[
            pl.BlockSpec((1, _RB), lambda i: (0, i)),
            pl.BlockSpec((_NC, _RB), lambda i: (0, i)),
            pl.BlockSpec((1024, 10), lambda i: (0, 0)),
            pl.BlockSpec((1, 8), lambda i: (0, 0)),
        ],
        out_shape=[
            jax.ShapeDtypeStruct((1, _NPAD), jnp.float32),
            jax.ShapeDtypeStruct((_NC, _NPAD), jnp.float32),
            jax.ShapeDtypeStruct((1024, 10), jnp.float32),
            jax.ShapeDtypeStruct((1, 8), jnp.float32),
        ],
    )(train_embeddings, concept, train_embedding, clusters, W_hx,
      b_hx.reshape(1, 10))

    knn, _, _ = _sc_topk()(norms, dots)

    l1_new = jnp.sum(knn[:, 0]) * (1.0 / (_K * _NC))
    return (y_pred, scal[0, 0], scal[0, 1], l1_new, scal[0, 2])
